# Initial kernel scaffold; baseline (speedup 1.0000x reference)
import jax
import jax.numpy as jnp
from jax.experimental import pallas as pl


def kernel(x, edge_index, edge_attr, ecc_w1, ecc_b1, ecc_w2, ecc_b2, ecc_root, ecc_bias, gat_w, gat_a_src, gat_a_dst, gat_bias, fc_w, fc_b, out_w, out_b):
    raise NotImplementedError("write your pallas kernel here")


# speedup vs baseline: 3.1212x; 3.1212x over previous
import jax
import jax.numpy as jnp
from jax import lax
from jax.experimental import pallas as pl
from jax.experimental.pallas import tpu as pltpu
from jax.experimental.pallas import tpu_sc as plsc

N = 10000
E = 160000
DF = 256
DE = 16
F = 64
KH = 32
FC = 32

NC = 2
NS = 16
NW = NC * NS
CH = 128
NFULL = 39
EW_BIG = 5008
EW_SMALL = 4992
ZROWS = 624
FE = 128
FP = 128

_mesh = plsc.VectorSubcoreMesh(core_axis_name="c", subcore_axis_name="s")
_sc_params = pltpu.CompilerParams(needs_layout_passes=False)


def _worker(c, s):
    w = s * NC + c
    base = jnp.where(w < 16, w * EW_BIG, 16 * EW_BIG + (w - 16) * EW_SMALL)
    return w, base


def _zero_fill(buf, rows, cols):
    z16 = jnp.zeros((16,), jnp.float32)

    def body(i, _):
        for j in range(cols // 16):
            buf[i, pl.ds(j * 16, 16)] = z16
        return 0

    lax.fori_loop(0, rows, body, 0)


def _zero_shared(s, zb_v, sh):
    _zero_fill(zb_v, 128, sh.shape[1])
    for j in range(4):
        pltpu.sync_copy(zb_v, sh.at[pl.ds(s * ZROWS + j * 128, 128)])
    pltpu.sync_copy(zb_v.at[pl.ds(0, 112)], sh.at[pl.ds(s * ZROWS + 512, 112)])

    @pl.when(s == NS - 1)
    def _():
        pltpu.sync_copy(zb_v.at[pl.ds(0, 16)], sh.at[pl.ds(NS * ZROWS, 16)])


def _writeback_shared(c, s, sh, out_hbm):
    pltpu.sync_copy(sh.at[pl.ds(s * ZROWS, ZROWS)],
                    out_hbm.at[c, pl.ds(s * ZROWS, ZROWS)])

    @pl.when(s == NS - 1)
    def _():
        pltpu.sync_copy(sh.at[pl.ds(NS * ZROWS, 16)],
                        out_hbm.at[c, pl.ds(NS * ZROWS, 16)])


def _s1_body(x_hbm, src_hbm, out_hbm, idx_v, rows_v, idx16_v, sem):
    c = lax.axis_index("c")
    s = lax.axis_index("s")
    w, base = _worker(c, s)

    def chunk(i, _):
        off = base + i * CH
        pltpu.sync_copy(src_hbm.at[pl.ds(off, CH)], idx_v)
        pltpu.async_copy(x_hbm.at[idx_v], rows_v, sem).wait()
        pltpu.sync_copy(rows_v, out_hbm.at[pl.ds(off, CH)])
        return 0

    lax.fori_loop(0, NFULL, chunk, 0)

    @pl.when(w < 16)
    def _():
        off = base + NFULL * CH
        pltpu.sync_copy(src_hbm.at[pl.ds(off, 16)], idx16_v)
        pltpu.async_copy(x_hbm.at[idx16_v], rows_v.at[pl.ds(0, 16)], sem).wait()
        pltpu.sync_copy(rows_v.at[pl.ds(0, 16)], out_hbm.at[pl.ds(off, 16)])


_DH = DF // 2

_s1 = pl.kernel(
    _s1_body,
    out_type=jax.ShapeDtypeStruct((E, _DH), jnp.int32),
    mesh=_mesh,
    compiler_params=_sc_params,
    scratch_types=[
        pltpu.VMEM((CH,), jnp.int32),
        pltpu.VMEM((CH, _DH), jnp.int32),
        pltpu.VMEM((16,), jnp.int32),
        pltpu.SemaphoreType.DMA,
    ],
)


def _t1_body(ea_ref, xs_ref, w1_ref, b1_ref, w2cat_ref, m_ref):
    h = jnp.maximum(
        jnp.dot(ea_ref[...], w1_ref[...], preferred_element_type=jnp.float32)
        + b1_ref[...],
        0.0,
    )
    y = jnp.dot(xs_ref[...], w2cat_ref[...], preferred_element_type=jnp.float32)
    acc = y[:, KH * F:]
    for k in range(KH):
        acc = acc + h[:, k:k + 1] * y[:, k * F:(k + 1) * F]
    m_ref[...] = jnp.concatenate(
        [acc, jnp.zeros((_BE, FP - F), jnp.float32)], axis=1)


_BE = 640


def _t1(edge_attr, xs, w1, b1, w2cat):
    return pl.pallas_call(
        _t1_body,
        grid=(E // _BE,),
        in_specs=[
            pl.BlockSpec((_BE, DE), lambda i: (i, 0)),
            pl.BlockSpec((_BE, DF), lambda i: (i, 0)),
            pl.BlockSpec((DE, KH), lambda i: (0, 0)),
            pl.BlockSpec((1, KH), lambda i: (0, 0)),
            pl.BlockSpec((DF, (KH + 1) * F), lambda i: (0, 0)),
        ],
        out_specs=pl.BlockSpec((_BE, FP), lambda i: (i, 0)),
        out_shape=jax.ShapeDtypeStruct((E, FP), jnp.float32),
    )(edge_attr, xs, w1, b1, w2cat)


def _s2_body(m_hbm, dst_hbm, out_hbm, didx_v, rows_v, didx16_v, agg_sh, sem):
    c = lax.axis_index("c")
    s = lax.axis_index("s")
    w, base = _worker(c, s)

    _zero_shared(s, rows_v, agg_sh)
    plsc.subcore_barrier()

    def chunk(i, _):
        off = base + i * CH
        pltpu.sync_copy(dst_hbm.at[pl.ds(off, CH)], didx_v)
        pltpu.sync_copy(m_hbm.at[pl.ds(off, CH)], rows_v)
        pltpu.sync_copy(rows_v, agg_sh.at[didx_v], add=True)
        return 0

    lax.fori_loop(0, NFULL, chunk, 0)

    @pl.when(w < 16)
    def _():
        off = base + NFULL * CH
        pltpu.sync_copy(dst_hbm.at[pl.ds(off, 16)], didx16_v)
        pltpu.sync_copy(m_hbm.at[pl.ds(off, 16)], rows_v.at[pl.ds(0, 16)])
        pltpu.sync_copy(rows_v.at[pl.ds(0, 16)], agg_sh.at[didx16_v], add=True)

    plsc.subcore_barrier()
    _writeback_shared(c, s, agg_sh, out_hbm)


_s2 = pl.kernel(
    _s2_body,
    out_type=jax.ShapeDtypeStruct((NC, N, FP), jnp.float32),
    mesh=_mesh,
    compiler_params=_sc_params,
    scratch_types=[
        pltpu.VMEM((CH,), jnp.int32),
        pltpu.VMEM((CH, FP), jnp.float32),
        pltpu.VMEM((16,), jnp.int32),
        pltpu.VMEM_SHARED((N, FP), jnp.float32),
        pltpu.SemaphoreType.DMA,
    ],
)


def _t2_body(x_ref, root_ref, agg_ref, ebias_ref, gw_ref, gas_ref, gad_ref,
             hgext_ref, as_ref, ad_ref):
    xr = jnp.dot(x_ref[...], root_ref[...], preferred_element_type=jnp.float32)
    x1 = jnp.maximum(
        agg_ref[0][:, :F] + agg_ref[1][:, :F] + xr + ebias_ref[...], 0.0)
    hg = jnp.dot(x1, gw_ref[...], preferred_element_type=jnp.float32)
    hgext_ref[...] = jnp.concatenate(
        [hg, jnp.ones((N, FE - F), jnp.float32)], axis=1)
    as_ref[...] = jnp.dot(hg, gas_ref[...], preferred_element_type=jnp.float32)
    ad_ref[...] = jnp.dot(hg, gad_ref[...], preferred_element_type=jnp.float32)


def _t2(x, root, agg, ebias, gw, gas, gad):
    return pl.pallas_call(
        _t2_body,
        out_shape=(
            jax.ShapeDtypeStruct((N, FE), jnp.float32),
            jax.ShapeDtypeStruct((N, 1), jnp.float32),
            jax.ShapeDtypeStruct((N, 1), jnp.float32),
        ),
    )(x, root, agg, ebias, gw, gas, gad)


def _s3_body(hgext_hbm, as_hbm, ad_hbm, src_hbm, dst_hbm, out_hbm,
             asv, adv, sidx_v, didx_v, hrows_v, exb_v, didx16_v, u_sh, sem):
    c = lax.axis_index("c")
    s = lax.axis_index("s")
    w, base = _worker(c, s)

    _zero_shared(s, hrows_v, u_sh)
    pltpu.sync_copy(as_hbm, asv)
    pltpu.sync_copy(ad_hbm, adv)
    plsc.subcore_barrier()

    def do_chunk(off, n_edges, didx):
        pltpu.sync_copy(src_hbm.at[pl.ds(off, n_edges)], sidx_v.at[pl.ds(0, n_edges)])
        pltpu.sync_copy(dst_hbm.at[pl.ds(off, n_edges)], didx)
        pltpu.async_copy(hgext_hbm.at[sidx_v.at[pl.ds(0, n_edges)]],
                         hrows_v.at[pl.ds(0, n_edges)], sem).wait()
        for g in range(n_edges // 16):
            s16 = sidx_v[pl.ds(g * 16, 16)]
            d16 = didx[pl.ds(g * 16, 16)]
            z = plsc.load_gather(asv, [s16]) + plsc.load_gather(adv, [d16])
            lr = jnp.where(z >= 0.0, z, 0.2 * z)
            exb_v[pl.ds(g * 16, 16)] = jnp.exp(lr)

        for e in range(n_edges):
            exs = plsc.load_gather(exb_v, [jnp.full((16,), e, jnp.int32)])
            for f in range(FE // 16):
                hrows_v[e, pl.ds(f * 16, 16)] = exs * hrows_v[e, pl.ds(f * 16, 16)]

        pltpu.sync_copy(hrows_v.at[pl.ds(0, n_edges)], u_sh.at[didx], add=True)

    def chunk(i, _):
        do_chunk(base + i * CH, CH, didx_v)
        return 0

    lax.fori_loop(0, NFULL, chunk, 0)

    @pl.when(w < 16)
    def _():
        do_chunk(base + NFULL * CH, 16, didx16_v)

    plsc.subcore_barrier()
    _writeback_shared(c, s, u_sh, out_hbm)


_s3 = pl.kernel(
    _s3_body,
    out_type=jax.ShapeDtypeStruct((NC, N, FE), jnp.float32),
    mesh=_mesh,
    compiler_params=_sc_params,
    scratch_types=[
        pltpu.VMEM((N,), jnp.float32),
        pltpu.VMEM((N,), jnp.float32),
        pltpu.VMEM((CH,), jnp.int32),
        pltpu.VMEM((CH,), jnp.int32),
        pltpu.VMEM((CH, FE), jnp.float32),
        pltpu.VMEM((CH,), jnp.float32),
        pltpu.VMEM((16,), jnp.int32),
        pltpu.VMEM_SHARED((N, FE), jnp.float32),
        pltpu.SemaphoreType.DMA,
    ],
)


def _t3_body(u_ref, gbias_ref, fcw_ref, fcb_ref, ow_ref, ob_ref, out_ref):
    u = u_ref[0] + u_ref[1]
    denom = u[:, F:F + 1]
    x2 = jnp.maximum(u[:, :F] / (denom + 1e-9) + gbias_ref[...], 0.0)
    pooled = jnp.mean(x2, axis=0, keepdims=True)
    fc = jnp.maximum(
        jnp.dot(pooled, fcw_ref[...], preferred_element_type=jnp.float32)
        + fcb_ref[...], 0.0)
    z = jnp.dot(fc, ow_ref[...], preferred_element_type=jnp.float32) + ob_ref[...]
    out_ref[...] = jax.nn.sigmoid(z)


def _t3(u, gbias, fcw, fcb, ow, ob):
    return pl.pallas_call(
        _t3_body,
        out_shape=jax.ShapeDtypeStruct((1, 1), jnp.float32),
    )(u, gbias, fcw, fcb, ow, ob)


def kernel(x, edge_index, edge_attr, ecc_w1, ecc_b1, ecc_w2, ecc_b2, ecc_root,
           ecc_bias, gat_w, gat_a_src, gat_a_dst, gat_bias, fc_w, fc_b, out_w,
           out_b):
    src = edge_index[0]
    dst = edge_index[1]
    x_i32 = jax.lax.bitcast_convert_type(
        x.astype(jnp.bfloat16).reshape(N, _DH, 2), jnp.int32)
    w2cat = jnp.concatenate(
        [jnp.transpose(ecc_w2, (1, 0, 2)).reshape(DF, KH * F), ecc_b2],
        axis=1).astype(jnp.bfloat16)

    xs_i32 = _s1(x_i32, src)
    xs = jax.lax.bitcast_convert_type(xs_i32, jnp.bfloat16).reshape(E, DF)
    m = _t1(edge_attr, xs, ecc_w1, ecc_b1.reshape(1, KH), w2cat)
    agg = _s2(m, dst)
    hgext, a_s, a_d = _t2(x, ecc_root, agg, ecc_bias.reshape(1, F), gat_w,
                          gat_a_src.reshape(F, 1), gat_a_dst.reshape(F, 1))
    u = _s3(hgext, a_s.reshape(N), a_d.reshape(N), src, dst)
    out = _t3(u, gat_bias.reshape(1, F), fc_w, fc_b.reshape(1, FC), out_w,
              out_b.reshape(1, 1))
    return out.reshape(1)


# speedup vs baseline: 6.4433x; 2.0643x over previous
import jax
import jax.numpy as jnp
from jax import lax
from jax.experimental import pallas as pl
from jax.experimental.pallas import tpu as pltpu
from jax.experimental.pallas import tpu_sc as plsc

N = 10000
E = 160000
DF = 256
DE = 16
F = 64
KH = 32
FC = 32

NC = 2
NS = 16
NW = NC * NS
EW = 4992
ZROWS = 624
FE = 128
FP = 128

_mesh = plsc.VectorSubcoreMesh(core_axis_name="c", subcore_axis_name="s")
_sc_params = pltpu.CompilerParams(needs_layout_passes=False)


def _worker(c, s):
    w = s * NC + c
    base = jnp.where(w < 16, w * (EW + 16), 16 * (EW + 16) + (w - 16) * EW)
    return w, base


def _vcopy_idx(src_ref, src_off, dst_ref, n):
    for i in range(n // 16):
        dst_ref[pl.ds(i * 16, 16)] = src_ref[pl.ds(src_off + i * 16, 16)]


def _zero_fill(buf, rows, cols):
    z16 = jnp.zeros((16,), jnp.float32)

    def body(i, _):
        for j in range(cols // 16):
            buf[i, pl.ds(j * 16, 16)] = z16
        return 0

    lax.fori_loop(0, rows, body, 0)


def _zero_shared(s, zb_v, sh):
    rows = zb_v.shape[0]
    _zero_fill(zb_v, rows, sh.shape[1])
    for j in range(ZROWS // rows):
        pltpu.sync_copy(zb_v, sh.at[pl.ds(s * ZROWS + j * rows, rows)])
    rem = ZROWS % rows
    if rem:
        pltpu.sync_copy(zb_v.at[pl.ds(0, rem)],
                        sh.at[pl.ds(s * ZROWS + ZROWS - rem, rem)])

    @pl.when(s == NS - 1)
    def _():
        pltpu.sync_copy(zb_v.at[pl.ds(0, 16)], sh.at[pl.ds(NS * ZROWS, 16)])


def _writeback_shared(c, s, sh, out_hbm):
    pltpu.sync_copy(sh.at[pl.ds(s * ZROWS, ZROWS)],
                    out_hbm.at[c, pl.ds(s * ZROWS, ZROWS)])

    @pl.when(s == NS - 1)
    def _():
        pltpu.sync_copy(sh.at[pl.ds(NS * ZROWS, 16)],
                        out_hbm.at[c, pl.ds(NS * ZROWS, 16)])


_C1 = 48
_NCH1 = EW // _C1


def _s1_body(x_hbm, src_hbm, out_hbm, idx_all, b0, b1, b2, b3, idx16,
             sg0, sg1, sg2, sg3, sw0, sw1, sw2, sw3):
    c = lax.axis_index("c")
    s = lax.axis_index("s")
    w, base = _worker(c, s)
    bufs = (b0, b1, b2, b3)
    sgs = (sg0, sg1, sg2, sg3)
    sws = (sw0, sw1, sw2, sw3)

    pltpu.sync_copy(src_hbm.at[pl.ds(base, EW)], idx_all)

    def start_gather(j, r, sem):
        pltpu.async_copy(x_hbm.at[idx_all.at[pl.ds(j * _C1, _C1)]],
                         bufs[r], sem)

    def wait_gather(r, sem):
        pltpu.make_async_copy(x_hbm.at[idx_all.at[pl.ds(0, _C1)]],
                              bufs[r], sem).wait()

    start_gather(0, 0, sgs[0])
    start_gather(1, 1, sgs[1])

    def it(g, _):
        for r in range(4):
            j = 4 * g + r
            r2 = (r + 2) % 4

            @pl.when(j + 2 < _NCH1)
            def _():
                @pl.when(j >= 2)
                def _():
                    pltpu.make_async_copy(
                        bufs[r2], out_hbm.at[pl.ds(base, _C1)], sws[r2]).wait()
                start_gather(j + 2, r2, sgs[r2])

            wait_gather(r, sgs[r])
            pltpu.async_copy(bufs[r], out_hbm.at[pl.ds(base + j * _C1, _C1)],
                             sws[r])
        return 0

    lax.fori_loop(0, _NCH1 // 4, it, 0)
    for r in range(4):
        pltpu.make_async_copy(bufs[r], out_hbm.at[pl.ds(base, _C1)],
                              sws[r]).wait()

    @pl.when(w < 16)
    def _():
        off = base + EW
        pltpu.sync_copy(src_hbm.at[pl.ds(off, 16)], idx16)
        pltpu.async_copy(x_hbm.at[idx16], b0.at[pl.ds(0, 16)], sg0).wait()
        pltpu.sync_copy(b0.at[pl.ds(0, 16)], out_hbm.at[pl.ds(off, 16)])


_s1 = pl.kernel(
    _s1_body,
    out_type=jax.ShapeDtypeStruct((E, DF), jnp.float32),
    mesh=_mesh,
    compiler_params=_sc_params,
    scratch_types=[
        pltpu.VMEM((EW,), jnp.int32),
        pltpu.VMEM((_C1, DF), jnp.float32),
        pltpu.VMEM((_C1, DF), jnp.float32),
        pltpu.VMEM((_C1, DF), jnp.float32),
        pltpu.VMEM((_C1, DF), jnp.float32),
        pltpu.VMEM((16,), jnp.int32),
    ] + [pltpu.SemaphoreType.DMA] * 8,
)


def _t1_body(ea_ref, xs_ref, w1_ref, b1_ref, w2cat_ref, s_ref, t_ref, m_ref):
    h = jnp.maximum(
        jnp.dot(ea_ref[...], w1_ref[...], preferred_element_type=jnp.float32)
        + b1_ref[...],
        0.0,
    )
    h_ext = jnp.concatenate([h, jnp.ones((_BE, 1), jnp.float32)],
                            axis=1).astype(jnp.bfloat16)
    xb = xs_ref[...].astype(jnp.bfloat16)
    y = jnp.dot(xb, w2cat_ref[...], preferred_element_type=jnp.float32)
    hexp = jnp.dot(h_ext, s_ref[...], preferred_element_type=jnp.float32)
    p = (hexp * y).astype(jnp.bfloat16)
    acc = jnp.dot(p, t_ref[...], preferred_element_type=jnp.float32)
    m_ref[...] = jnp.concatenate(
        [acc, jnp.zeros((_BE, FP - F), jnp.float32)], axis=1)


_BE = 640


def _t1(edge_attr, xs, w1, b1, w2cat, s_mat, t_mat):
    return pl.pallas_call(
        _t1_body,
        grid=(E // _BE,),
        in_specs=[
            pl.BlockSpec((_BE, DE), lambda i: (i, 0)),
            pl.BlockSpec((_BE, DF), lambda i: (i, 0)),
            pl.BlockSpec((DE, KH), lambda i: (0, 0)),
            pl.BlockSpec((1, KH), lambda i: (0, 0)),
            pl.BlockSpec((DF, (KH + 1) * F), lambda i: (0, 0)),
            pl.BlockSpec((KH + 1, (KH + 1) * F), lambda i: (0, 0)),
            pl.BlockSpec(((KH + 1) * F, F), lambda i: (0, 0)),
        ],
        out_specs=pl.BlockSpec((_BE, FP), lambda i: (i, 0)),
        out_shape=jax.ShapeDtypeStruct((E, FP), jnp.float32),
    )(edge_attr, xs, w1, b1, w2cat, s_mat, t_mat)


def _s2_body(m_hbm, dst_hbm, out_hbm, didx_all, b0, b1, b2, b3,
             d0, d1, d2, d3, didx16, agg_sh,
             sr0, sr1, sr2, sr3, sa0, sa1, sa2, sa3):
    c = lax.axis_index("c")
    s = lax.axis_index("s")
    w, base = _worker(c, s)
    bufs = (b0, b1, b2, b3)
    dbufs = (d0, d1, d2, d3)
    srs = (sr0, sr1, sr2, sr3)
    sas = (sa0, sa1, sa2, sa3)

    _zero_shared(s, b0, agg_sh)
    pltpu.sync_copy(dst_hbm.at[pl.ds(base, EW)], didx_all)
    plsc.subcore_barrier()

    def start_read(j, r, sem):
        pltpu.async_copy(m_hbm.at[pl.ds(base + j * _C1, _C1)], bufs[r], sem)

    start_read(0, 0, srs[0])
    start_read(1, 1, srs[1])

    def it(g, _):
        for r in range(4):
            j = 4 * g + r
            r2 = (r + 2) % 4

            @pl.when(j + 2 < _NCH1)
            def _():
                @pl.when(j >= 2)
                def _():
                    pltpu.make_async_copy(
                        bufs[r2], agg_sh.at[dbufs[r2]], sas[r2]).wait()
                start_read(j + 2, r2, srs[r2])

            pltpu.make_async_copy(m_hbm.at[pl.ds(base, _C1)], bufs[r],
                                  srs[r]).wait()
            _vcopy_idx(didx_all, j * _C1, dbufs[r], _C1)
            pltpu.async_copy(bufs[r], agg_sh.at[dbufs[r]], sas[r], add=True)
        return 0

    lax.fori_loop(0, _NCH1 // 4, it, 0)
    for r in range(4):
        pltpu.make_async_copy(bufs[r], agg_sh.at[dbufs[r]], sas[r]).wait()

    @pl.when(w < 16)
    def _():
        off = base + EW
        pltpu.sync_copy(dst_hbm.at[pl.ds(off, 16)], didx16)
        pltpu.sync_copy(m_hbm.at[pl.ds(off, 16)], b0.at[pl.ds(0, 16)])
        pltpu.sync_copy(b0.at[pl.ds(0, 16)], agg_sh.at[didx16], add=True)

    plsc.subcore_barrier()
    _writeback_shared(c, s, agg_sh, out_hbm)


_s2 = pl.kernel(
    _s2_body,
    out_type=jax.ShapeDtypeStruct((NC, N, FP), jnp.float32),
    mesh=_mesh,
    compiler_params=_sc_params,
    scratch_types=[
        pltpu.VMEM((EW,), jnp.int32),
        pltpu.VMEM((_C1, FP), jnp.float32),
        pltpu.VMEM((_C1, FP), jnp.float32),
        pltpu.VMEM((_C1, FP), jnp.float32),
        pltpu.VMEM((_C1, FP), jnp.float32),
        pltpu.VMEM((_C1,), jnp.int32),
        pltpu.VMEM((_C1,), jnp.int32),
        pltpu.VMEM((_C1,), jnp.int32),
        pltpu.VMEM((_C1,), jnp.int32),
        pltpu.VMEM((16,), jnp.int32),
        pltpu.VMEM_SHARED((N, FP), jnp.float32),
    ] + [pltpu.SemaphoreType.DMA] * 8,
)


def _t2_body(x_ref, root_ref, agg_ref, ebias_ref, gw_ref, gas_ref, gad_ref,
             hgext_ref, as_ref, ad_ref):
    xr = jnp.dot(x_ref[...], root_ref[...], preferred_element_type=jnp.float32)
    x1 = jnp.maximum(
        agg_ref[0][:, :F] + agg_ref[1][:, :F] + xr + ebias_ref[...], 0.0)
    hg = jnp.dot(x1, gw_ref[...], preferred_element_type=jnp.float32)
    hgext_ref[...] = jnp.concatenate(
        [hg, jnp.ones((N, 1), jnp.float32),
         jnp.zeros((N, FE - F - 1), jnp.float32)], axis=1)
    as_ref[...] = jnp.dot(hg, gas_ref[...], preferred_element_type=jnp.float32)
    ad_ref[...] = jnp.dot(hg, gad_ref[...], preferred_element_type=jnp.float32)


def _t2(x, root, agg, ebias, gw, gas, gad):
    return pl.pallas_call(
        _t2_body,
        out_shape=(
            jax.ShapeDtypeStruct((N, FE), jnp.float32),
            jax.ShapeDtypeStruct((N, 1), jnp.float32),
            jax.ShapeDtypeStruct((N, 1), jnp.float32),
        ),
    )(x, root, agg, ebias, gw, gas, gad)


_C3 = 64
_NCH3 = EW // _C3


def _s3_body(hgext_hbm, as_hbm, ad_hbm, src_hbm, dst_hbm, out_hbm,
             asv, adv, sidx_all, didx_all, b0, b1, exb, dbuf,
             sidx16, didx16, u_sh, sg0, sg1):
    c = lax.axis_index("c")
    s = lax.axis_index("s")
    w, base = _worker(c, s)

    _zero_shared(s, b0, u_sh)
    pltpu.sync_copy(as_hbm, asv)
    pltpu.sync_copy(ad_hbm, adv)
    pltpu.sync_copy(src_hbm.at[pl.ds(base, EW)], sidx_all)
    pltpu.sync_copy(dst_hbm.at[pl.ds(base, EW)], didx_all)
    plsc.subcore_barrier()

    def start_gather(j, buf, sem):
        pltpu.async_copy(hgext_hbm.at[sidx_all.at[pl.ds(j * _C3, _C3)]],
                         buf, sem)

    def wait_gather(buf, sem):
        pltpu.make_async_copy(hgext_hbm.at[sidx_all.at[pl.ds(0, _C3)]],
                              buf, sem).wait()

    def process(j, buf):
        for g in range(_C3 // 16):
            s16 = sidx_all[pl.ds(j * _C3 + g * 16, 16)]
            d16 = didx_all[pl.ds(j * _C3 + g * 16, 16)]
            z = plsc.load_gather(asv, [s16]) + plsc.load_gather(adv, [d16])
            lr = jnp.where(z >= 0.0, z, 0.2 * z)
            exb[pl.ds(g * 16, 16)] = jnp.exp(lr)
        for e in range(_C3):
            exs = plsc.load_gather(exb, [jnp.full((16,), e, jnp.int32)])
            for f in range(5):
                buf[e, pl.ds(f * 16, 16)] = exs * buf[e, pl.ds(f * 16, 16)]
        _vcopy_idx(didx_all, j * _C3, dbuf, _C3)
        pltpu.sync_copy(buf, u_sh.at[dbuf], add=True)

    start_gather(0, b0, sg0)

    def it(g, _):
        j0 = 2 * g
        wait_gather(b0, sg0)
        start_gather(j0 + 1, b1, sg1)
        process(j0, b0)
        wait_gather(b1, sg1)

        @pl.when(g < _NCH3 // 2 - 1)
        def _():
            start_gather(j0 + 2, b0, sg0)

        process(j0 + 1, b1)
        return 0

    lax.fori_loop(0, _NCH3 // 2, it, 0)

    @pl.when(w < 16)
    def _():
        off = base + EW
        pltpu.sync_copy(src_hbm.at[pl.ds(off, 16)], sidx16)
        pltpu.sync_copy(dst_hbm.at[pl.ds(off, 16)], didx16)
        pltpu.async_copy(hgext_hbm.at[sidx16], b0.at[pl.ds(0, 16)], sg0).wait()
        s16 = sidx16[...]
        d16 = didx16[...]
        z = plsc.load_gather(asv, [s16]) + plsc.load_gather(adv, [d16])
        lr = jnp.where(z >= 0.0, z, 0.2 * z)
        exb[pl.ds(0, 16)] = jnp.exp(lr)
        for e in range(16):
            exs = plsc.load_gather(exb, [jnp.full((16,), e, jnp.int32)])
            for f in range(5):
                b0[e, pl.ds(f * 16, 16)] = exs * b0[e, pl.ds(f * 16, 16)]
        pltpu.sync_copy(b0.at[pl.ds(0, 16)], u_sh.at[didx16], add=True)

    plsc.subcore_barrier()
    _writeback_shared(c, s, u_sh, out_hbm)


_s3 = pl.kernel(
    _s3_body,
    out_type=jax.ShapeDtypeStruct((NC, N, FE), jnp.float32),
    mesh=_mesh,
    compiler_params=_sc_params,
    scratch_types=[
        pltpu.VMEM((N,), jnp.float32),
        pltpu.VMEM((N,), jnp.float32),
        pltpu.VMEM((EW,), jnp.int32),
        pltpu.VMEM((EW,), jnp.int32),
        pltpu.VMEM((_C3, FE), jnp.float32),
        pltpu.VMEM((_C3, FE), jnp.float32),
        pltpu.VMEM((_C3,), jnp.float32),
        pltpu.VMEM((_C3,), jnp.int32),
        pltpu.VMEM((16,), jnp.int32),
        pltpu.VMEM((16,), jnp.int32),
        pltpu.VMEM_SHARED((N, FE), jnp.float32),
        pltpu.SemaphoreType.DMA,
        pltpu.SemaphoreType.DMA,
    ],
)


def _t3_body(u_ref, gbias_ref, fcw_ref, fcb_ref, ow_ref, ob_ref, out_ref):
    u = u_ref[0] + u_ref[1]
    denom = u[:, F:F + 1]
    x2 = jnp.maximum(u[:, :F] / (denom + 1e-9) + gbias_ref[...], 0.0)
    pooled = jnp.mean(x2, axis=0, keepdims=True)
    fc = jnp.maximum(
        jnp.dot(pooled, fcw_ref[...], preferred_element_type=jnp.float32)
        + fcb_ref[...], 0.0)
    z = jnp.dot(fc, ow_ref[...], preferred_element_type=jnp.float32) + ob_ref[...]
    out_ref[...] = jax.nn.sigmoid(z)


def _t3(u, gbias, fcw, fcb, ow, ob):
    return pl.pallas_call(
        _t3_body,
        out_shape=jax.ShapeDtypeStruct((1, 1), jnp.float32),
    )(u, gbias, fcw, fcb, ow, ob)


def kernel(x, edge_index, edge_attr, ecc_w1, ecc_b1, ecc_w2, ecc_b2, ecc_root,
           ecc_bias, gat_w, gat_a_src, gat_a_dst, gat_bias, fc_w, fc_b, out_w,
           out_b):
    src = edge_index[0]
    dst = edge_index[1]
    w2cat = jnp.concatenate(
        [jnp.transpose(ecc_w2, (1, 0, 2)).reshape(DF, KH * F), ecc_b2],
        axis=1).astype(jnp.bfloat16)

    xs = _s1(x, src)
    s_mat = jnp.repeat(jnp.eye(KH + 1, dtype=jnp.bfloat16), F, axis=1)
    t_mat = jnp.tile(jnp.eye(F, dtype=jnp.bfloat16), (KH + 1, 1))
    m = _t1(edge_attr, xs, ecc_w1, ecc_b1.reshape(1, KH), w2cat, s_mat, t_mat)
    agg = _s2(m, dst)
    hgext, a_s, a_d = _t2(x, ecc_root, agg, ecc_bias.reshape(1, F), gat_w,
                          gat_a_src.reshape(F, 1), gat_a_dst.reshape(F, 1))
    u = _s3(hgext, a_s.reshape(N), a_d.reshape(N), src, dst)
    out = _t3(u, gat_bias.reshape(1, F), fc_w, fc_b.reshape(1, FC), out_w,
              out_b.reshape(1, 1))
    return out.reshape(1)
